# baseline (speedup 1.0000x reference)
import jax
import jax.numpy as jnp
from jax import lax
from jax.experimental import pallas as pl
from jax.experimental.pallas import tpu as pltpu
from jax.experimental.pallas import tpu_sc as plsc

N = 10000
E = 320000
IN_DIM = 128
HID = 16
OUT_DIM = 7
NPAD = 10240
NC = 2
NS = 16
NW = NC * NS
EPT = E // NW
CHUNK = 2000
NCH = EPT // CHUNK
RPT = NPAD // NS

_mesh = plsc.VectorSubcoreMesh(
    core_axis_name="c", subcore_axis_name="s", num_cores=NC, num_subcores=NS
)
_sc_params = pltpu.CompilerParams(use_tc_tiling_on_sc=False)


def _deg_body(ei_hbm, out_hbm, idx2, ones_v, stg, deg_sh, sem):
    c = lax.axis_index("c")
    s = lax.axis_index("s")
    wid = c * NS + s

    def fill(i, _):
        ones_v[pl.ds(i * 16, 16)] = jnp.full((16,), 1.0, jnp.float32)
        stg[pl.ds(i * 16, 16)] = jnp.zeros((16,), jnp.float32)
        return 0

    lax.fori_loop(0, CHUNK // 16, fill, 0)
    descs = [
        pltpu.async_copy(
            ei_hbm.at[1, pl.ds(wid * EPT + j * CHUNK, CHUNK)], idx2.at[j], sem)
        for j in range(NCH)
    ]
    pltpu.sync_copy(stg.at[pl.ds(0, RPT)], deg_sh.at[pl.ds(s * RPT, RPT)])
    for d in descs:
        d.wait()
    plsc.subcore_barrier()
    for j in range(NCH):
        pltpu.sync_copy(ones_v, deg_sh.at[idx2.at[j]], add=True)
    plsc.subcore_barrier()
    pltpu.sync_copy(deg_sh.at[pl.ds(s * RPT, RPT)], stg.at[pl.ds(0, RPT)])
    pltpu.sync_copy(stg.at[pl.ds(0, RPT)], out_hbm.at[pl.ds(c * NPAD + s * RPT, RPT)])


_deg_call = pl.kernel(
    _deg_body,
    out_type=jax.ShapeDtypeStruct((2 * NPAD,), jnp.float32),
    mesh=_mesh,
    scratch_types=[
        pltpu.VMEM((NCH, CHUNK), jnp.int32),
        pltpu.VMEM((CHUNK,), jnp.float32),
        pltpu.VMEM((CHUNK,), jnp.float32),
        pltpu.VMEM_SHARED((NPAD,), jnp.float32),
        pltpu.SemaphoreType.DMA,
    ],
    compiler_params=_sc_params,
)


def _msg_body(tab_hbm, ei_hbm, out_hbm, idx_s2, idx_d2, rows0, rows1, stg,
              acc_sh, semi, sg0, sg1, ssc):
    c = lax.axis_index("c")
    s = lax.axis_index("s")
    wid = c * NS + s

    def zfill(i, _):
        stg[i, :] = jnp.zeros((16,), jnp.float32)
        return 0

    lax.fori_loop(0, RPT, zfill, 0)
    descs = []
    for j in range(NCH):
        base = wid * EPT + j * CHUNK
        descs.append(pltpu.async_copy(
            ei_hbm.at[0, pl.ds(base, CHUNK)], idx_s2.at[j], semi))
        descs.append(pltpu.async_copy(
            ei_hbm.at[1, pl.ds(base, CHUNK)], idx_d2.at[j], semi))
    pltpu.sync_copy(stg, acc_sh.at[pl.ds(s * RPT, RPT)])
    for d in descs:
        d.wait()
    plsc.subcore_barrier()

    bufs = (rows0, rows1)
    sems = (sg0, sg1)
    g = [None] * NCH
    sc = [None] * NCH
    g[0] = pltpu.async_copy(tab_hbm.at[idx_s2.at[0]], bufs[0], sems[0])
    for j in range(NCH):
        g[j].wait()
        if j >= 1:
            sc[j - 1].wait()
        if j + 1 < NCH:
            g[j + 1] = pltpu.async_copy(
                tab_hbm.at[idx_s2.at[j + 1]], bufs[(j + 1) % 2], sems[(j + 1) % 2])
        sc[j] = pltpu.async_copy(
            bufs[j % 2], acc_sh.at[idx_d2.at[j]], ssc, add=True)
    sc[NCH - 1].wait()
    plsc.subcore_barrier()
    pltpu.sync_copy(acc_sh.at[pl.ds(s * RPT, RPT)], stg)
    pltpu.sync_copy(stg, out_hbm.at[pl.ds(c * NPAD + s * RPT, RPT)])


_msg_call = pl.kernel(
    _msg_body,
    out_type=jax.ShapeDtypeStruct((2 * NPAD, HID), jnp.float32),
    mesh=_mesh,
    scratch_types=[
        pltpu.VMEM((NCH, CHUNK), jnp.int32),
        pltpu.VMEM((NCH, CHUNK), jnp.int32),
        pltpu.VMEM((CHUNK, HID), jnp.float32),
        pltpu.VMEM((CHUNK, HID), jnp.float32),
        pltpu.VMEM((RPT, HID), jnp.float32),
        pltpu.VMEM_SHARED((NPAD, HID), jnp.float32),
        pltpu.SemaphoreType.DMA,
        pltpu.SemaphoreType.DMA,
        pltpu.SemaphoreType.DMA,
        pltpu.SemaphoreType.DMA,
    ],
    compiler_params=_sc_params,
)


GRID = 2
RB = NPAD // GRID
RBP = RB * HID // 128
DBP = RB // 128
GRID3 = 10
RB3 = NPAD // GRID3
RBP3 = RB3 * HID // 128
DBP3 = RB3 // 128


def _dscale_blk(dp_ref, rbp, dbp):
    deg = dp_ref[0] + dp_ref[1] + 1.0
    dinv = lax.rsqrt(deg)
    r_col = lax.broadcasted_iota(jnp.int32, (rbp, dbp), 0)
    q_row = lax.broadcasted_iota(jnp.int32, (rbp, dbp), 1)
    qoh = jnp.where(r_col // 16 == q_row, 1.0, 0.0)
    u = jnp.dot(qoh, dinv, preferred_element_type=jnp.float32)
    r2 = lax.broadcasted_iota(jnp.int32, (rbp, 128), 0)
    l2 = lax.broadcasted_iota(jnp.int32, (rbp, 128), 1)
    cols = []
    for a in range(8):
        loh = jnp.where(l2 == 8 * (r2 % 16) + a, 1.0, 0.0)
        cols.append(jnp.sum(u * loh, axis=1, keepdims=True))
    dvec8 = jnp.concatenate(cols, axis=1)
    a_row = lax.broadcasted_iota(jnp.int32, (8, 128), 0)
    m_col = lax.broadcasted_iota(jnp.int32, (8, 128), 1)
    bcast = jnp.where(m_col // 16 == a_row, 1.0, 0.0)
    return jnp.dot(dvec8, bcast, preferred_element_type=jnp.float32)


def _s1a_body(x3_ref, w1a_ref, o_ref):
    x3 = x3_ref[...]
    acc = jnp.zeros((RBP, 128), jnp.float32)
    for a in range(8):
        acc = acc + jnp.dot(x3[:, a, :], w1a_ref[a],
                            preferred_element_type=jnp.float32)
    o_ref[...] = acc


def _s1b_body(h_ref, dp_ref, o_ref):
    o_ref[...] = h_ref[...] * _dscale_blk(dp_ref, RBP, DBP)


def _s2_body(p_ref, h1s_ref, dp_ref, b1_ref, w2bd_ref, o_ref):
    dscale = _dscale_blk(dp_ref, RBP, DBP)
    m = p_ref[0] + p_ref[1] + h1s_ref[...]
    a = jnp.maximum(m * dscale + b1_ref[...], 0.0)
    h2 = jnp.dot(a, w2bd_ref[...], preferred_element_type=jnp.float32)
    o_ref[...] = h2 * dscale


def _s3_body(p_ref, h2s_ref, dp_ref, b2_ref, gs_ref, o_ref):
    dscale = _dscale_blk(dp_ref, RBP3, DBP3)
    z = (p_ref[0] + p_ref[1] + h2s_ref[...]) * dscale + b2_ref[...]
    mcol = lax.broadcasted_iota(jnp.int32, (RBP3, 128), 1)
    mask = mcol % HID < OUT_DIM
    zm = jnp.where(mask, z, jnp.float32(-1e30))
    mx = jnp.max(zm, axis=1, keepdims=True)
    e = jnp.where(mask, jnp.exp(z - mx), 0.0)
    gs = jnp.dot(e, gs_ref[...], preferred_element_type=jnp.float32)
    lse = jnp.log(gs) + mx
    res = z - lse
    n_i = lax.broadcasted_iota(jnp.int32, (RB3, RBP3), 0)
    r_i = lax.broadcasted_iota(jnp.int32, (RB3, RBP3), 1)
    m_i = lax.broadcasted_iota(jnp.int32, (128, OUT_DIM), 0)
    j_i = lax.broadcasted_iota(jnp.int32, (128, OUT_DIM), 1)
    out = jnp.zeros((RB3, OUT_DIM), jnp.float32)
    for a in range(8):
        rowexp = jnp.where(n_i == 8 * r_i + a, 1.0, 0.0)
        colsel = jnp.where(m_i == HID * a + j_i, 1.0, 0.0)
        t = jnp.dot(res, colsel, preferred_element_type=jnp.float32)
        out = out + jnp.dot(rowexp, t, preferred_element_type=jnp.float32)
    o_ref[...] = out


def kernel(x, edge_index, W1, b1, W2, b2):
    ei = edge_index.astype(jnp.int32)
    W2p = jnp.pad(W2, ((0, 0), (0, HID - OUT_DIM)))
    W1a = jnp.stack([jnp.pad(W1, ((0, 0), (HID * a, 128 - HID * a - HID)))
                     for a in range(8)])
    aeq = (jnp.arange(128)[:, None] // HID) == (jnp.arange(128)[None, :] // HID)
    W2bd = jnp.where(aeq, jnp.tile(W2p, (8, 8)), 0.0)
    gs_mat = aeq.astype(jnp.float32)
    b1r = jnp.reshape(jnp.tile(b1, 8), (1, 128))
    b2p = jnp.reshape(jnp.tile(jnp.pad(b2, (0, HID - OUT_DIM)), 8), (1, 128))
    x3 = jnp.reshape(x, (N // 8, 8, IN_DIM))

    deg_pk = _deg_call(ei).reshape(2, NPAD // 128, 128)

    h1_pk = pl.pallas_call(
        _s1a_body,
        grid=(GRID,),
        in_specs=[
            pl.BlockSpec((RBP, 8, IN_DIM), lambda i: (i, 0, 0)),
            pl.BlockSpec((8, IN_DIM, 128), lambda i: (0, 0, 0)),
        ],
        out_specs=pl.BlockSpec((RBP, 128), lambda i: (i, 0)),
        out_shape=jax.ShapeDtypeStruct((NPAD * HID // 128, 128), jnp.float32),
    )(x3, W1a)

    h1s_pk = pl.pallas_call(
        _s1b_body,
        grid=(GRID,),
        in_specs=[
            pl.BlockSpec((RBP, 128), lambda i: (i, 0)),
            pl.BlockSpec((2, DBP, 128), lambda i: (0, i, 0)),
        ],
        out_specs=pl.BlockSpec((RBP, 128), lambda i: (i, 0)),
        out_shape=jax.ShapeDtypeStruct((NPAD * HID // 128, 128), jnp.float32),
    )(h1_pk, deg_pk)

    p1_pk = _msg_call(h1s_pk.reshape(NPAD, HID), ei).reshape(
        2, NPAD * HID // 128, 128)

    h2s_pk = pl.pallas_call(
        _s2_body,
        grid=(GRID,),
        in_specs=[
            pl.BlockSpec((2, RBP, 128), lambda i: (0, i, 0)),
            pl.BlockSpec((RBP, 128), lambda i: (i, 0)),
            pl.BlockSpec((2, DBP, 128), lambda i: (0, i, 0)),
            pl.BlockSpec((1, 128), lambda i: (0, 0)),
            pl.BlockSpec((128, 128), lambda i: (0, 0)),
        ],
        out_specs=pl.BlockSpec((RBP, 128), lambda i: (i, 0)),
        out_shape=jax.ShapeDtypeStruct((NPAD * HID // 128, 128), jnp.float32),
    )(p1_pk, h1s_pk, deg_pk, b1r, W2bd)

    p2_pk = _msg_call(h2s_pk.reshape(NPAD, HID), ei).reshape(
        2, NPAD * HID // 128, 128)

    out = pl.pallas_call(
        _s3_body,
        grid=(GRID3,),
        in_specs=[
            pl.BlockSpec((2, RBP3, 128), lambda i: (0, i, 0)),
            pl.BlockSpec((RBP3, 128), lambda i: (i, 0)),
            pl.BlockSpec((2, DBP3, 128), lambda i: (0, i, 0)),
            pl.BlockSpec((1, 128), lambda i: (0, 0)),
            pl.BlockSpec((128, 128), lambda i: (0, 0)),
        ],
        out_specs=pl.BlockSpec((RB3, OUT_DIM), lambda i: (i, 0)),
        out_shape=jax.ShapeDtypeStruct((N, OUT_DIM), jnp.float32),
    )(p2_pk, h2s_pk, deg_pk, b2p, gs_mat)

    return out


# speedup vs baseline: 1.0543x; 1.0543x over previous
import jax
import jax.numpy as jnp
from jax import lax
from jax.experimental import pallas as pl
from jax.experimental.pallas import tpu as pltpu
from jax.experimental.pallas import tpu_sc as plsc

N = 10000
E = 320000
IN_DIM = 128
HID = 16
OUT_DIM = 7
NPAD = 10240
GSZ = NPAD // 8
NC = 2
NS = 16
NW = NC * NS
EPT = E // NW
CHUNK = 2000
NCH = EPT // CHUNK
RPT = NPAD // NS

_mesh = plsc.VectorSubcoreMesh(
    core_axis_name="c", subcore_axis_name="s", num_cores=NC, num_subcores=NS
)
_sc_params = pltpu.CompilerParams(use_tc_tiling_on_sc=False)


def _to_vorder(idx2, j):
    def body(k, _):
        t = idx2[j, pl.ds(k * 16, 16)]
        q = lax.shift_right_logical(
            lax.shift_right_logical(t, 8) * 13108, 16)
        idx2[j, pl.ds(k * 16, 16)] = t * 8 - q * 10239
        return 0

    lax.fori_loop(0, CHUNK // 16, body, 0)


def _deg_body(ei_hbm, out_hbm, idx2, ones_v, stg, deg_sh, sem):
    c = lax.axis_index("c")
    s = lax.axis_index("s")
    wid = c * NS + s

    def fill(i, _):
        ones_v[pl.ds(i * 16, 16)] = jnp.full((16,), 1.0, jnp.float32)
        stg[pl.ds(i * 16, 16)] = jnp.zeros((16,), jnp.float32)
        return 0

    lax.fori_loop(0, CHUNK // 16, fill, 0)
    descs = [
        pltpu.async_copy(
            ei_hbm.at[pl.ds(E + wid * EPT + j * CHUNK, CHUNK)], idx2.at[j], sem)
        for j in range(NCH)
    ]
    pltpu.sync_copy(stg.at[pl.ds(0, RPT)], deg_sh.at[pl.ds(s * RPT, RPT)])
    for d in descs:
        d.wait()
    plsc.subcore_barrier()
    _to_vorder(idx2, 0)
    for j in range(NCH):
        d = pltpu.async_copy(ones_v, deg_sh.at[idx2.at[j]], sem, add=True)
        if j + 1 < NCH:
            _to_vorder(idx2, j + 1)
        d.wait()
    plsc.subcore_barrier()
    pltpu.sync_copy(deg_sh.at[pl.ds(s * RPT, RPT)], stg.at[pl.ds(0, RPT)])
    pltpu.sync_copy(stg.at[pl.ds(0, RPT)], out_hbm.at[pl.ds(c * NPAD + s * RPT, RPT)])


_deg_call = pl.kernel(
    _deg_body,
    out_type=jax.ShapeDtypeStruct((2 * NPAD,), jnp.float32),
    mesh=_mesh,
    scratch_types=[
        pltpu.VMEM((NCH, CHUNK), jnp.int32),
        pltpu.VMEM((CHUNK,), jnp.float32),
        pltpu.VMEM((CHUNK,), jnp.float32),
        pltpu.VMEM_SHARED((NPAD,), jnp.float32),
        pltpu.SemaphoreType.DMA,
    ],
    compiler_params=_sc_params,
)


def _msg_body(tab_hbm, ei_hbm, out_hbm, idx_s2, idx_d2, rows0, rows1, stg,
              acc_sh, semi, sg0, sg1, ssc):
    c = lax.axis_index("c")
    s = lax.axis_index("s")
    wid = c * NS + s

    def zfill(i, _):
        stg[i, :] = jnp.zeros((16,), jnp.float32)
        return 0

    lax.fori_loop(0, RPT, zfill, 0)
    descs = []
    for j in range(NCH):
        base = wid * EPT + j * CHUNK
        descs.append(pltpu.async_copy(
            ei_hbm.at[pl.ds(base, CHUNK)], idx_s2.at[j], semi))
        descs.append(pltpu.async_copy(
            ei_hbm.at[pl.ds(E + base, CHUNK)], idx_d2.at[j], semi))
    pltpu.sync_copy(stg, acc_sh.at[pl.ds(s * RPT, RPT)])
    for d in descs:
        d.wait()
    plsc.subcore_barrier()

    bufs = (rows0, rows1)
    sems = (sg0, sg1)
    _to_vorder(idx_s2, 0)
    _to_vorder(idx_d2, 0)
    g = [None] * NCH
    sc = [None] * NCH
    g[0] = pltpu.async_copy(tab_hbm.at[idx_s2.at[0]], bufs[0], sems[0])
    for j in range(NCH):
        if j + 1 < NCH:
            _to_vorder(idx_s2, j + 1)
            _to_vorder(idx_d2, j + 1)
        g[j].wait()
        if j >= 1:
            sc[j - 1].wait()
        if j + 1 < NCH:
            g[j + 1] = pltpu.async_copy(
                tab_hbm.at[idx_s2.at[j + 1]], bufs[(j + 1) % 2], sems[(j + 1) % 2])
        sc[j] = pltpu.async_copy(
            bufs[j % 2], acc_sh.at[idx_d2.at[j]], ssc, add=True)
    sc[NCH - 1].wait()
    plsc.subcore_barrier()
    pltpu.sync_copy(acc_sh.at[pl.ds(s * RPT, RPT)], stg)
    pltpu.sync_copy(stg, out_hbm.at[pl.ds(c * NPAD + s * RPT, RPT)])


_msg_call = pl.kernel(
    _msg_body,
    out_type=jax.ShapeDtypeStruct((2 * NPAD, HID), jnp.float32),
    mesh=_mesh,
    scratch_types=[
        pltpu.VMEM((NCH, CHUNK), jnp.int32),
        pltpu.VMEM((NCH, CHUNK), jnp.int32),
        pltpu.VMEM((CHUNK, HID), jnp.float32),
        pltpu.VMEM((CHUNK, HID), jnp.float32),
        pltpu.VMEM((RPT, HID), jnp.float32),
        pltpu.VMEM_SHARED((NPAD, HID), jnp.float32),
        pltpu.SemaphoreType.DMA,
        pltpu.SemaphoreType.DMA,
        pltpu.SemaphoreType.DMA,
        pltpu.SemaphoreType.DMA,
    ],
    compiler_params=_sc_params,
)


GRID = 2
RBP = GSZ // GRID
DBP = NPAD // 128 // GRID
PKR = GSZ
DEGR = NPAD // 128


def _dscale_blk(dp_ref, rbp, dbp):
    deg = dp_ref[0] + dp_ref[1] + 1.0
    dinv = lax.rsqrt(deg)
    r_col = lax.broadcasted_iota(jnp.int32, (rbp, dbp), 0)
    q_row = lax.broadcasted_iota(jnp.int32, (rbp, dbp), 1)
    qoh = jnp.where(r_col // 16 == q_row, 1.0, 0.0)
    u = jnp.dot(qoh, dinv, preferred_element_type=jnp.float32)
    r2 = lax.broadcasted_iota(jnp.int32, (rbp, 128), 0)
    l2 = lax.broadcasted_iota(jnp.int32, (rbp, 128), 1)
    cols = []
    for a in range(8):
        loh = jnp.where(l2 == 8 * (r2 % 16) + a, 1.0, 0.0)
        cols.append(jnp.sum(u * loh, axis=1, keepdims=True))
    dvec8 = jnp.concatenate(cols, axis=1)
    a_row = lax.broadcasted_iota(jnp.int32, (8, 128), 0)
    m_col = lax.broadcasted_iota(jnp.int32, (8, 128), 1)
    bcast = jnp.where(m_col // 16 == a_row, 1.0, 0.0)
    return jnp.dot(dvec8, bcast, preferred_element_type=jnp.float32)


def _s1a_body(x_ref, w_ref, o_ref):
    for a in range(8):
        rows = min(GSZ, N - GSZ * a)
        h = jnp.dot(x_ref[pl.ds(GSZ * a, rows), :], w_ref[...],
                    preferred_element_type=jnp.float32)
        o_ref[pl.ds(0, rows), pl.ds(a * HID, HID)] = h


def _s1b_body(h_ref, dp_ref, o_ref):
    o_ref[...] = h_ref[...] * _dscale_blk(dp_ref, RBP, DBP)


def _s2_body(p_ref, h1s_ref, dp_ref, b1_ref, w2bd_ref, o_ref):
    dscale = _dscale_blk(dp_ref, RBP, DBP)
    m = p_ref[0] + p_ref[1] + h1s_ref[...]
    a = jnp.maximum(m * dscale + b1_ref[...], 0.0)
    h2 = jnp.dot(a, w2bd_ref[...], preferred_element_type=jnp.float32)
    o_ref[...] = h2 * dscale


def _s3_body(p_ref, h2s_ref, dp_ref, b2_ref, o_ref):
    dinv = lax.rsqrt(dp_ref[0] + dp_ref[1] + 1.0)
    r_col = lax.broadcasted_iota(jnp.int32, (PKR, DEGR), 0)
    q_row = lax.broadcasted_iota(jnp.int32, (PKR, DEGR), 1)
    qoh = jnp.where(r_col // 16 == q_row, 1.0, 0.0)
    u = jnp.dot(qoh, dinv, preferred_element_type=jnp.float32)
    r2 = lax.broadcasted_iota(jnp.int32, (PKR, 128), 0)
    l2 = lax.broadcasted_iota(jnp.int32, (PKR, 128), 1)
    mcol = lax.broadcasted_iota(jnp.int32, (PKR, HID), 1)
    mask = mcol < OUT_DIM
    ps = p_ref[0] + p_ref[1] + h2s_ref[...]
    for a in range(8):
        loh = jnp.where(l2 == 8 * (r2 % 16) + a, 1.0, 0.0)
        dvec = jnp.sum(u * loh, axis=1, keepdims=True)
        z = ps[:, a * HID:(a + 1) * HID] * dvec + b2_ref[...]
        zm = jnp.where(mask, z, jnp.float32(-1e30))
        mx = jnp.max(zm, axis=1, keepdims=True)
        e = jnp.where(mask, jnp.exp(z - mx), 0.0)
        lse = jnp.log(jnp.sum(e, axis=1, keepdims=True)) + mx
        rows = min(GSZ, N - GSZ * a)
        o_ref[pl.ds(GSZ * a, rows), :] = (z - lse)[:rows, :OUT_DIM]


def kernel(x, edge_index, W1, b1, W2, b2):
    ei = edge_index.reshape(2 * E).astype(jnp.int32)
    W2p = jnp.pad(W2, ((0, 0), (0, HID - OUT_DIM)))
    aeq = (jnp.arange(128)[:, None] // HID) == (jnp.arange(128)[None, :] // HID)
    W2bd = jnp.where(aeq, jnp.tile(W2p, (8, 8)), 0.0)
    b1r = jnp.reshape(jnp.tile(b1, 8), (1, 128))
    b2p = jnp.reshape(jnp.pad(b2, (0, HID - OUT_DIM)), (1, HID))

    deg_pk = _deg_call(ei).reshape(2, DEGR, 128)

    h1_pk = pl.pallas_call(
        _s1a_body,
        in_specs=[
            pl.BlockSpec((N, IN_DIM), lambda: (0, 0)),
            pl.BlockSpec((IN_DIM, HID), lambda: (0, 0)),
        ],
        out_specs=pl.BlockSpec((GSZ, 128), lambda: (0, 0)),
        out_shape=jax.ShapeDtypeStruct((GSZ, 128), jnp.float32),
    )(x, W1)

    h1s_pk = pl.pallas_call(
        _s1b_body,
        grid=(GRID,),
        in_specs=[
            pl.BlockSpec((RBP, 128), lambda i: (i, 0)),
            pl.BlockSpec((2, DBP, 128), lambda i: (0, i, 0)),
        ],
        out_specs=pl.BlockSpec((RBP, 128), lambda i: (i, 0)),
        out_shape=jax.ShapeDtypeStruct((GSZ, 128), jnp.float32),
    )(h1_pk, deg_pk)

    p1_pk = _msg_call(h1s_pk.reshape(NPAD, HID), ei).reshape(2, GSZ, 128)

    h2s_pk = pl.pallas_call(
        _s2_body,
        grid=(GRID,),
        in_specs=[
            pl.BlockSpec((2, RBP, 128), lambda i: (0, i, 0)),
            pl.BlockSpec((RBP, 128), lambda i: (i, 0)),
            pl.BlockSpec((2, DBP, 128), lambda i: (0, i, 0)),
            pl.BlockSpec((1, 128), lambda i: (0, 0)),
            pl.BlockSpec((128, 128), lambda i: (0, 0)),
        ],
        out_specs=pl.BlockSpec((RBP, 128), lambda i: (i, 0)),
        out_shape=jax.ShapeDtypeStruct((GSZ, 128), jnp.float32),
    )(p1_pk, h1s_pk, deg_pk, b1r, W2bd)

    p2_pk = _msg_call(h2s_pk.reshape(NPAD, HID), ei).reshape(2, GSZ, 128)

    out = pl.pallas_call(
        _s3_body,
        in_specs=[
            pl.BlockSpec((2, GSZ, 128), lambda: (0, 0, 0)),
            pl.BlockSpec((GSZ, 128), lambda: (0, 0)),
            pl.BlockSpec((2, DEGR, 128), lambda: (0, 0, 0)),
            pl.BlockSpec((1, HID), lambda: (0, 0)),
        ],
        out_specs=pl.BlockSpec((N, OUT_DIM), lambda: (0, 0)),
        out_shape=jax.ShapeDtypeStruct((N, OUT_DIM), jnp.float32),
    )(p2_pk, h2s_pk, deg_pk, b2p)

    return out


# speedup vs baseline: 1.1244x; 1.0665x over previous
import jax
import jax.numpy as jnp
from jax import lax
from jax.experimental import pallas as pl
from jax.experimental.pallas import tpu as pltpu
from jax.experimental.pallas import tpu_sc as plsc

N = 10000
E = 320000
IN_DIM = 128
HID = 16
OUT_DIM = 7
NPAD = 10240
GSZ = NPAD // 8
NC = 2
NS = 16
NW = NC * NS
EPT = E // NW
CHUNK = 2000
NCH = EPT // CHUNK
RPT = NPAD // NS

_mesh = plsc.VectorSubcoreMesh(
    core_axis_name="c", subcore_axis_name="s", num_cores=NC, num_subcores=NS
)
_sc_params = pltpu.CompilerParams(use_tc_tiling_on_sc=False)


def _to_vorder_one(idx2, j, k):
    t = idx2[j, pl.ds(k * 16, 16)]
    q = lax.shift_right_logical(lax.shift_right_logical(t, 8) * 13108, 16)
    idx2[j, pl.ds(k * 16, 16)] = t * 8 - q * 10239


def _to_vorder(idx2, j):
    def body(k, _):
        _to_vorder_one(idx2, j, k)
        return 0

    lax.fori_loop(0, CHUNK // 16, body, 0)


def _deg_body(ei_hbm, out_hbm, idx2, ones_v, stg, deg_sh, sem):
    c = lax.axis_index("c")
    s = lax.axis_index("s")
    wid = c * NS + s

    def fill(i, _):
        ones_v[pl.ds(i * 16, 16)] = jnp.full((16,), 1.0, jnp.float32)
        stg[pl.ds(i * 16, 16)] = jnp.zeros((16,), jnp.float32)
        return 0

    lax.fori_loop(0, CHUNK // 16, fill, 0)
    descs = [
        pltpu.async_copy(
            ei_hbm.at[pl.ds(E + wid * EPT + j * CHUNK, CHUNK)], idx2.at[j], sem)
        for j in range(NCH)
    ]
    pltpu.sync_copy(stg.at[pl.ds(0, RPT)], deg_sh.at[pl.ds(s * RPT, RPT)])
    for d in descs:
        d.wait()
    plsc.subcore_barrier()
    _to_vorder(idx2, 0)
    for j in range(NCH):
        d = pltpu.async_copy(ones_v, deg_sh.at[idx2.at[j]], sem, add=True)
        if j + 1 < NCH:
            _to_vorder(idx2, j + 1)
        d.wait()
    plsc.subcore_barrier()
    pltpu.sync_copy(deg_sh.at[pl.ds(s * RPT, RPT)], stg.at[pl.ds(0, RPT)])
    pltpu.sync_copy(stg.at[pl.ds(0, RPT)], out_hbm.at[pl.ds(c * NPAD + s * RPT, RPT)])


_deg_call = pl.kernel(
    _deg_body,
    out_type=jax.ShapeDtypeStruct((2 * NPAD,), jnp.float32),
    mesh=_mesh,
    scratch_types=[
        pltpu.VMEM((NCH, CHUNK), jnp.int32),
        pltpu.VMEM((CHUNK,), jnp.float32),
        pltpu.VMEM((CHUNK,), jnp.float32),
        pltpu.VMEM_SHARED((NPAD,), jnp.float32),
        pltpu.SemaphoreType.DMA,
    ],
    compiler_params=_sc_params,
)


def _msg_body(tab_hbm, ei_hbm, out_hbm, idx_s2, idx_d2, rows0, rows1, stg,
              acc_sh, semi, sg0, sg1, ssc):
    c = lax.axis_index("c")
    s = lax.axis_index("s")
    wid = c * NS + s

    def zfill(i, _):
        stg[i, :] = jnp.zeros((16,), jnp.float32)
        return 0

    lax.fori_loop(0, RPT, zfill, 0)
    descs = []
    for j in range(NCH):
        base = wid * EPT + j * CHUNK
        descs.append(pltpu.async_copy(
            ei_hbm.at[pl.ds(base, CHUNK)], idx_s2.at[j], semi))
        descs.append(pltpu.async_copy(
            ei_hbm.at[pl.ds(E + base, CHUNK)], idx_d2.at[j], semi))
    pltpu.sync_copy(stg, acc_sh.at[pl.ds(s * RPT, RPT)])
    descs[0].wait()
    _to_vorder(idx_s2, 0)
    for d in descs[1:]:
        d.wait()
    plsc.subcore_barrier()

    bufs = (rows0, rows1)
    sems = (sg0, sg1)
    g = [None] * NCH
    sc = [None] * NCH
    g[0] = pltpu.async_copy(tab_hbm.at[idx_s2.at[0]], bufs[0], sems[0])
    _to_vorder(idx_d2, 0)
    for j in range(NCH):
        if j + 1 < NCH:
            _to_vorder(idx_s2, j + 1)
            _to_vorder(idx_d2, j + 1)
        g[j].wait()
        if j >= 1:
            sc[j - 1].wait()
        if j + 1 < NCH:
            g[j + 1] = pltpu.async_copy(
                tab_hbm.at[idx_s2.at[j + 1]], bufs[(j + 1) % 2], sems[(j + 1) % 2])
        sc[j] = pltpu.async_copy(
            bufs[j % 2], acc_sh.at[idx_d2.at[j]], ssc, add=True)
    sc[NCH - 1].wait()
    plsc.subcore_barrier()
    pltpu.sync_copy(acc_sh.at[pl.ds(s * RPT, RPT)], stg)
    pltpu.sync_copy(stg, out_hbm.at[pl.ds(c * NPAD + s * RPT, RPT)])


_msg_call = pl.kernel(
    _msg_body,
    out_type=jax.ShapeDtypeStruct((2 * NPAD, HID), jnp.float32),
    mesh=_mesh,
    scratch_types=[
        pltpu.VMEM((NCH, CHUNK), jnp.int32),
        pltpu.VMEM((NCH, CHUNK), jnp.int32),
        pltpu.VMEM((CHUNK, HID), jnp.float32),
        pltpu.VMEM((CHUNK, HID), jnp.float32),
        pltpu.VMEM((RPT, HID), jnp.float32),
        pltpu.VMEM_SHARED((NPAD, HID), jnp.float32),
        pltpu.SemaphoreType.DMA,
        pltpu.SemaphoreType.DMA,
        pltpu.SemaphoreType.DMA,
        pltpu.SemaphoreType.DMA,
    ],
    compiler_params=_sc_params,
)


GRID = 2
RBP = GSZ // GRID
DBP = NPAD // 128 // GRID
PKR = GSZ
DEGR = NPAD // 128


def _dscale_blk(dp_ref, rbp, dbp):
    deg = dp_ref[0] + dp_ref[1] + 1.0
    dinv = lax.rsqrt(deg)
    r_col = lax.broadcasted_iota(jnp.int32, (rbp, dbp), 0)
    q_row = lax.broadcasted_iota(jnp.int32, (rbp, dbp), 1)
    qoh = jnp.where(r_col // 16 == q_row, 1.0, 0.0)
    u = jnp.dot(qoh, dinv, preferred_element_type=jnp.float32)
    r2 = lax.broadcasted_iota(jnp.int32, (rbp, 128), 0)
    l2 = lax.broadcasted_iota(jnp.int32, (rbp, 128), 1)
    cols = []
    for a in range(8):
        loh = jnp.where(l2 == 8 * (r2 % 16) + a, 1.0, 0.0)
        cols.append(jnp.sum(u * loh, axis=1, keepdims=True))
    dvec8 = jnp.concatenate(cols, axis=1)
    a_row = lax.broadcasted_iota(jnp.int32, (8, 128), 0)
    m_col = lax.broadcasted_iota(jnp.int32, (8, 128), 1)
    bcast = jnp.where(m_col // 16 == a_row, 1.0, 0.0)
    return jnp.dot(dvec8, bcast, preferred_element_type=jnp.float32)


def _s1a_body(x_ref, w_ref, o_ref):
    for a in range(8):
        rows = min(GSZ, N - GSZ * a)
        h = jnp.dot(x_ref[pl.ds(GSZ * a, rows), :], w_ref[...],
                    preferred_element_type=jnp.float32)
        o_ref[pl.ds(0, rows), pl.ds(a * HID, HID)] = h


def _s1b_body(h_ref, dp_ref, o_ref, ds_ref):
    dscale = _dscale_blk(dp_ref, RBP, DBP)
    ds_ref[...] = dscale
    o_ref[...] = h_ref[...] * dscale


def _s2_body(p_ref, h1s_ref, ds_ref, b1_ref, w2bd_ref, o_ref):
    dscale = ds_ref[...]
    m = p_ref[0] + p_ref[1] + h1s_ref[...]
    a = jnp.maximum(m * dscale + b1_ref[...], 0.0)
    h2 = jnp.dot(a, w2bd_ref[...], preferred_element_type=jnp.float32)
    o_ref[...] = h2 * dscale


def _s3_body(p_ref, h2s_ref, ds_ref, b2_ref, o_ref):
    mcol = lax.broadcasted_iota(jnp.int32, (PKR, HID), 1)
    mask = mcol < OUT_DIM
    ps = (p_ref[0] + p_ref[1] + h2s_ref[...]) * ds_ref[...]
    for a in range(8):
        z = ps[:, a * HID:(a + 1) * HID] + b2_ref[...]
        zm = jnp.where(mask, z, jnp.float32(-1e30))
        mx = jnp.max(zm, axis=1, keepdims=True)
        e = jnp.where(mask, jnp.exp(z - mx), 0.0)
        lse = jnp.log(jnp.sum(e, axis=1, keepdims=True)) + mx
        rows = min(GSZ, N - GSZ * a)
        o_ref[pl.ds(GSZ * a, rows), :] = (z - lse)[:rows, :OUT_DIM]


def kernel(x, edge_index, W1, b1, W2, b2):
    ei = edge_index.reshape(2 * E).astype(jnp.int32)
    W2p = jnp.pad(W2, ((0, 0), (0, HID - OUT_DIM)))
    aeq = (jnp.arange(128)[:, None] // HID) == (jnp.arange(128)[None, :] // HID)
    W2bd = jnp.where(aeq, jnp.tile(W2p, (8, 8)), 0.0)
    b1r = jnp.reshape(jnp.tile(b1, 8), (1, 128))
    b2p = jnp.reshape(jnp.pad(b2, (0, HID - OUT_DIM)), (1, HID))

    deg_pk = _deg_call(ei).reshape(2, DEGR, 128)

    h1_pk = pl.pallas_call(
        _s1a_body,
        in_specs=[
            pl.BlockSpec((N, IN_DIM), lambda: (0, 0)),
            pl.BlockSpec((IN_DIM, HID), lambda: (0, 0)),
        ],
        out_specs=pl.BlockSpec((GSZ, 128), lambda: (0, 0)),
        out_shape=jax.ShapeDtypeStruct((GSZ, 128), jnp.float32),
    )(x, W1)

    h1s_pk, dscale_pk = pl.pallas_call(
        _s1b_body,
        grid=(GRID,),
        in_specs=[
            pl.BlockSpec((RBP, 128), lambda i: (i, 0)),
            pl.BlockSpec((2, DBP, 128), lambda i: (0, i, 0)),
        ],
        out_specs=[
            pl.BlockSpec((RBP, 128), lambda i: (i, 0)),
            pl.BlockSpec((RBP, 128), lambda i: (i, 0)),
        ],
        out_shape=[
            jax.ShapeDtypeStruct((GSZ, 128), jnp.float32),
            jax.ShapeDtypeStruct((GSZ, 128), jnp.float32),
        ],
    )(h1_pk, deg_pk)

    p1_pk = _msg_call(h1s_pk.reshape(NPAD, HID), ei).reshape(2, GSZ, 128)

    h2s_pk = pl.pallas_call(
        _s2_body,
        grid=(GRID,),
        in_specs=[
            pl.BlockSpec((2, RBP, 128), lambda i: (0, i, 0)),
            pl.BlockSpec((RBP, 128), lambda i: (i, 0)),
            pl.BlockSpec((RBP, 128), lambda i: (i, 0)),
            pl.BlockSpec((1, 128), lambda i: (0, 0)),
            pl.BlockSpec((128, 128), lambda i: (0, 0)),
        ],
        out_specs=pl.BlockSpec((RBP, 128), lambda i: (i, 0)),
        out_shape=jax.ShapeDtypeStruct((GSZ, 128), jnp.float32),
    )(p1_pk, h1s_pk, dscale_pk, b1r, W2bd)

    p2_pk = _msg_call(h2s_pk.reshape(NPAD, HID), ei).reshape(2, GSZ, 128)

    out = pl.pallas_call(
        _s3_body,
        in_specs=[
            pl.BlockSpec((2, GSZ, 128), lambda: (0, 0, 0)),
            pl.BlockSpec((GSZ, 128), lambda: (0, 0)),
            pl.BlockSpec((GSZ, 128), lambda: (0, 0)),
            pl.BlockSpec((1, HID), lambda: (0, 0)),
        ],
        out_specs=pl.BlockSpec((N, OUT_DIM), lambda: (0, 0)),
        out_shape=jax.ShapeDtypeStruct((N, OUT_DIM), jnp.float32),
    )(p2_pk, h2s_pk, dscale_pk, b2p)

    return out


# speedup vs baseline: 1.1246x; 1.0002x over previous
import jax
import jax.numpy as jnp
from jax import lax
from jax.experimental import pallas as pl
from jax.experimental.pallas import tpu as pltpu
from jax.experimental.pallas import tpu_sc as plsc

N = 10000
E = 320000
IN_DIM = 128
HID = 16
OUT_DIM = 7
NPAD = 10240
GSZ = NPAD // 8
NC = 2
NS = 16
NW = NC * NS
EPT = E // NW
CHUNK = 2000
NCH = EPT // CHUNK
RPT = NPAD // NS

_mesh = plsc.VectorSubcoreMesh(
    core_axis_name="c", subcore_axis_name="s", num_cores=NC, num_subcores=NS
)
_sc_params = pltpu.CompilerParams(use_tc_tiling_on_sc=False)


def _to_vorder_one(idx2, j, k):
    t = idx2[j, pl.ds(k * 16, 16)]
    q = lax.shift_right_logical(lax.shift_right_logical(t, 8) * 13108, 16)
    idx2[j, pl.ds(k * 16, 16)] = t * 8 - q * 10239


def _to_vorder(idx2, j):
    def body(k, _):
        _to_vorder_one(idx2, j, k)
        return 0

    lax.fori_loop(0, CHUNK // 16, body, 0)


def _deg_body(ei_hbm, out_hbm, idx2, ones_v, stg, deg_sh, sem):
    c = lax.axis_index("c")
    s = lax.axis_index("s")
    wid = c * NS + s

    def fill(i, _):
        ones_v[pl.ds(i * 16, 16)] = jnp.full((16,), 1.0, jnp.float32)
        stg[pl.ds(i * 16, 16)] = jnp.zeros((16,), jnp.float32)
        return 0

    lax.fori_loop(0, CHUNK // 16, fill, 0)
    descs = [
        pltpu.async_copy(
            ei_hbm.at[pl.ds(E + wid * EPT + j * CHUNK, CHUNK)], idx2.at[j], sem)
        for j in range(NCH)
    ]
    pltpu.sync_copy(stg.at[pl.ds(0, RPT)], deg_sh.at[pl.ds(s * RPT, RPT)])
    for d in descs:
        d.wait()
    plsc.subcore_barrier()
    _to_vorder(idx2, 0)
    ds = [None] * NCH
    for j in range(NCH):
        ds[j] = pltpu.async_copy(ones_v, deg_sh.at[idx2.at[j]], sem, add=True)
        if j + 1 < NCH:
            _to_vorder(idx2, j + 1)
    for d in ds:
        d.wait()
    plsc.subcore_barrier()
    pltpu.sync_copy(deg_sh.at[pl.ds(s * RPT, RPT)], stg.at[pl.ds(0, RPT)])
    pltpu.sync_copy(stg.at[pl.ds(0, RPT)], out_hbm.at[pl.ds(c * NPAD + s * RPT, RPT)])


_deg_call = pl.kernel(
    _deg_body,
    out_type=jax.ShapeDtypeStruct((2 * NPAD,), jnp.float32),
    mesh=_mesh,
    scratch_types=[
        pltpu.VMEM((NCH, CHUNK), jnp.int32),
        pltpu.VMEM((CHUNK,), jnp.float32),
        pltpu.VMEM((CHUNK,), jnp.float32),
        pltpu.VMEM_SHARED((NPAD,), jnp.float32),
        pltpu.SemaphoreType.DMA,
    ],
    compiler_params=_sc_params,
)


def _msg_body(tab_hbm, ei_hbm, out_hbm, idx_s2, idx_d2, rows0, rows1, stg,
              acc_sh, semi, sg0, sg1, ssc):
    c = lax.axis_index("c")
    s = lax.axis_index("s")
    wid = c * NS + s

    def zfill(i, _):
        stg[i, :] = jnp.zeros((16,), jnp.float32)
        return 0

    lax.fori_loop(0, RPT, zfill, 0)
    descs = []
    for j in range(NCH):
        base = wid * EPT + j * CHUNK
        descs.append(pltpu.async_copy(
            ei_hbm.at[pl.ds(base, CHUNK)], idx_s2.at[j], semi))
        descs.append(pltpu.async_copy(
            ei_hbm.at[pl.ds(E + base, CHUNK)], idx_d2.at[j], semi))
    pltpu.sync_copy(stg, acc_sh.at[pl.ds(s * RPT, RPT)])
    descs[0].wait()
    _to_vorder(idx_s2, 0)
    for d in descs[1:]:
        d.wait()
    plsc.subcore_barrier()

    bufs = (rows0, rows1)
    sems = (sg0, sg1)
    g = [None] * NCH
    sc = [None] * NCH
    g[0] = pltpu.async_copy(tab_hbm.at[idx_s2.at[0]], bufs[0], sems[0])
    _to_vorder(idx_d2, 0)
    for j in range(NCH):
        if j + 1 < NCH:
            _to_vorder(idx_s2, j + 1)
            _to_vorder(idx_d2, j + 1)
        g[j].wait()
        if j >= 1:
            sc[j - 1].wait()
        if j + 1 < NCH:
            g[j + 1] = pltpu.async_copy(
                tab_hbm.at[idx_s2.at[j + 1]], bufs[(j + 1) % 2], sems[(j + 1) % 2])
        sc[j] = pltpu.async_copy(
            bufs[j % 2], acc_sh.at[idx_d2.at[j]], ssc, add=True)
    sc[NCH - 1].wait()
    plsc.subcore_barrier()
    pltpu.sync_copy(acc_sh.at[pl.ds(s * RPT, RPT)], stg)
    pltpu.sync_copy(stg, out_hbm.at[pl.ds(c * NPAD + s * RPT, RPT)])


_msg_call = pl.kernel(
    _msg_body,
    out_type=jax.ShapeDtypeStruct((2 * NPAD, HID), jnp.float32),
    mesh=_mesh,
    scratch_types=[
        pltpu.VMEM((NCH, CHUNK), jnp.int32),
        pltpu.VMEM((NCH, CHUNK), jnp.int32),
        pltpu.VMEM((CHUNK, HID), jnp.float32),
        pltpu.VMEM((CHUNK, HID), jnp.float32),
        pltpu.VMEM((RPT, HID), jnp.float32),
        pltpu.VMEM_SHARED((NPAD, HID), jnp.float32),
        pltpu.SemaphoreType.DMA,
        pltpu.SemaphoreType.DMA,
        pltpu.SemaphoreType.DMA,
        pltpu.SemaphoreType.DMA,
    ],
    compiler_params=_sc_params,
)


GRID = 2
RBP = GSZ // GRID
DBP = NPAD // 128 // GRID
PKR = GSZ
DEGR = NPAD // 128


def _dscale_blk(dp_ref, rbp, dbp):
    deg = dp_ref[0] + dp_ref[1] + 1.0
    dinv = lax.rsqrt(deg)
    r_col = lax.broadcasted_iota(jnp.int32, (rbp, dbp), 0)
    q_row = lax.broadcasted_iota(jnp.int32, (rbp, dbp), 1)
    qoh = jnp.where(r_col // 16 == q_row, 1.0, 0.0)
    u = jnp.dot(qoh, dinv, preferred_element_type=jnp.float32)
    r2 = lax.broadcasted_iota(jnp.int32, (rbp, 128), 0)
    l2 = lax.broadcasted_iota(jnp.int32, (rbp, 128), 1)
    cols = []
    for a in range(8):
        loh = jnp.where(l2 == 8 * (r2 % 16) + a, 1.0, 0.0)
        cols.append(jnp.sum(u * loh, axis=1, keepdims=True))
    dvec8 = jnp.concatenate(cols, axis=1)
    a_row = lax.broadcasted_iota(jnp.int32, (8, 128), 0)
    m_col = lax.broadcasted_iota(jnp.int32, (8, 128), 1)
    bcast = jnp.where(m_col // 16 == a_row, 1.0, 0.0)
    return jnp.dot(dvec8, bcast, preferred_element_type=jnp.float32)


def _s1a_body(x_ref, w_ref, o_ref):
    for a in range(8):
        rows = min(GSZ, N - GSZ * a)
        h = jnp.dot(x_ref[pl.ds(GSZ * a, rows), :], w_ref[...],
                    preferred_element_type=jnp.float32)
        o_ref[pl.ds(0, rows), pl.ds(a * HID, HID)] = h


def _s1b_body(h_ref, dp_ref, o_ref, ds_ref):
    dscale = _dscale_blk(dp_ref, RBP, DBP)
    ds_ref[...] = dscale
    o_ref[...] = h_ref[...] * dscale


def _s2_body(p_ref, h1s_ref, ds_ref, b1_ref, w2bd_ref, o_ref):
    dscale = ds_ref[...]
    m = p_ref[0] + p_ref[1] + h1s_ref[...]
    a = jnp.maximum(m * dscale + b1_ref[...], 0.0)
    h2 = jnp.dot(a, w2bd_ref[...], preferred_element_type=jnp.float32)
    o_ref[...] = h2 * dscale


def _s3_body(p_ref, h2s_ref, ds_ref, b2_ref, o_ref):
    mcol = lax.broadcasted_iota(jnp.int32, (PKR, HID), 1)
    mask = mcol < OUT_DIM
    ps = (p_ref[0] + p_ref[1] + h2s_ref[...]) * ds_ref[...]
    for a in range(8):
        z = ps[:, a * HID:(a + 1) * HID] + b2_ref[...]
        zm = jnp.where(mask, z, jnp.float32(-1e30))
        mx = jnp.max(zm, axis=1, keepdims=True)
        e = jnp.where(mask, jnp.exp(z - mx), 0.0)
        lse = jnp.log(jnp.sum(e, axis=1, keepdims=True)) + mx
        rows = min(GSZ, N - GSZ * a)
        o_ref[pl.ds(GSZ * a, rows), :] = (z - lse)[:rows, :OUT_DIM]


def kernel(x, edge_index, W1, b1, W2, b2):
    ei = edge_index.reshape(2 * E).astype(jnp.int32)
    W2p = jnp.pad(W2, ((0, 0), (0, HID - OUT_DIM)))
    aeq = (jnp.arange(128)[:, None] // HID) == (jnp.arange(128)[None, :] // HID)
    W2bd = jnp.where(aeq, jnp.tile(W2p, (8, 8)), 0.0)
    b1r = jnp.reshape(jnp.tile(b1, 8), (1, 128))
    b2p = jnp.reshape(jnp.pad(b2, (0, HID - OUT_DIM)), (1, HID))

    deg_pk = _deg_call(ei).reshape(2, DEGR, 128)

    h1_pk = pl.pallas_call(
        _s1a_body,
        in_specs=[
            pl.BlockSpec((N, IN_DIM), lambda: (0, 0)),
            pl.BlockSpec((IN_DIM, HID), lambda: (0, 0)),
        ],
        out_specs=pl.BlockSpec((GSZ, 128), lambda: (0, 0)),
        out_shape=jax.ShapeDtypeStruct((GSZ, 128), jnp.float32),
    )(x, W1)

    h1s_pk, dscale_pk = pl.pallas_call(
        _s1b_body,
        grid=(GRID,),
        in_specs=[
            pl.BlockSpec((RBP, 128), lambda i: (i, 0)),
            pl.BlockSpec((2, DBP, 128), lambda i: (0, i, 0)),
        ],
        out_specs=[
            pl.BlockSpec((RBP, 128), lambda i: (i, 0)),
            pl.BlockSpec((RBP, 128), lambda i: (i, 0)),
        ],
        out_shape=[
            jax.ShapeDtypeStruct((GSZ, 128), jnp.float32),
            jax.ShapeDtypeStruct((GSZ, 128), jnp.float32),
        ],
    )(h1_pk, deg_pk)

    p1_pk = _msg_call(h1s_pk.reshape(NPAD, HID), ei).reshape(2, GSZ, 128)

    h2s_pk = pl.pallas_call(
        _s2_body,
        grid=(GRID,),
        in_specs=[
            pl.BlockSpec((2, RBP, 128), lambda i: (0, i, 0)),
            pl.BlockSpec((RBP, 128), lambda i: (i, 0)),
            pl.BlockSpec((RBP, 128), lambda i: (i, 0)),
            pl.BlockSpec((1, 128), lambda i: (0, 0)),
            pl.BlockSpec((128, 128), lambda i: (0, 0)),
        ],
        out_specs=pl.BlockSpec((RBP, 128), lambda i: (i, 0)),
        out_shape=jax.ShapeDtypeStruct((GSZ, 128), jnp.float32),
    )(p1_pk, h1s_pk, dscale_pk, b1r, W2bd)

    p2_pk = _msg_call(h2s_pk.reshape(NPAD, HID), ei).reshape(2, GSZ, 128)

    out = pl.pallas_call(
        _s3_body,
        in_specs=[
            pl.BlockSpec((2, GSZ, 128), lambda: (0, 0, 0)),
            pl.BlockSpec((GSZ, 128), lambda: (0, 0)),
            pl.BlockSpec((GSZ, 128), lambda: (0, 0)),
            pl.BlockSpec((1, HID), lambda: (0, 0)),
        ],
        out_specs=pl.BlockSpec((N, OUT_DIM), lambda: (0, 0)),
        out_shape=jax.ShapeDtypeStruct((N, OUT_DIM), jnp.float32),
    )(p2_pk, h2s_pk, dscale_pk, b2p)

    return out


# speedup vs baseline: 1.1251x; 1.0005x over previous
import jax
import jax.numpy as jnp
from jax import lax
from jax.experimental import pallas as pl
from jax.experimental.pallas import tpu as pltpu
from jax.experimental.pallas import tpu_sc as plsc

N = 10000
E = 320000
IN_DIM = 128
HID = 16
OUT_DIM = 7
NPAD = 10240
GSZ = NPAD // 8
NC = 2
NS = 16
NW = NC * NS
EPT = E // NW
CHUNK = 2000
NCH = EPT // CHUNK
RPT = NPAD // NS

_mesh = plsc.VectorSubcoreMesh(
    core_axis_name="c", subcore_axis_name="s", num_cores=NC, num_subcores=NS
)
_sc_params = pltpu.CompilerParams(use_tc_tiling_on_sc=False)


def _to_vorder_one(idx2, j, k):
    t = idx2[j, pl.ds(k * 16, 16)]
    q = lax.shift_right_logical(lax.shift_right_logical(t, 8) * 13108, 16)
    idx2[j, pl.ds(k * 16, 16)] = t * 8 - q * 10239


def _to_vorder(idx2, j):
    def body(k, _):
        _to_vorder_one(idx2, j, k)
        return 0

    lax.fori_loop(0, CHUNK // 16, body, 0)


def _deg_body(ei_hbm, out_hbm, idx2, ones_v, stg, deg_sh, sem):
    c = lax.axis_index("c")
    s = lax.axis_index("s")
    wid = c * NS + s

    def fill(i, _):
        ones_v[pl.ds(i * 16, 16)] = jnp.full((16,), 1.0, jnp.float32)
        stg[pl.ds(i * 16, 16)] = jnp.zeros((16,), jnp.float32)
        return 0

    lax.fori_loop(0, CHUNK // 16, fill, 0)
    descs = [
        pltpu.async_copy(
            ei_hbm.at[pl.ds(E + wid * EPT + j * CHUNK, CHUNK)], idx2.at[j], sem)
        for j in range(NCH)
    ]
    pltpu.sync_copy(stg.at[pl.ds(0, RPT)], deg_sh.at[pl.ds(s * RPT, RPT)])
    for d in descs:
        d.wait()
    plsc.subcore_barrier()
    _to_vorder(idx2, 0)
    ds = [None] * NCH
    for j in range(NCH):
        ds[j] = pltpu.async_copy(ones_v, deg_sh.at[idx2.at[j]], sem, add=True)
        if j + 1 < NCH:
            _to_vorder(idx2, j + 1)
    for d in ds:
        d.wait()
    plsc.subcore_barrier()
    pltpu.sync_copy(deg_sh.at[pl.ds(s * RPT, RPT)], stg.at[pl.ds(0, RPT)])
    pltpu.sync_copy(stg.at[pl.ds(0, RPT)], out_hbm.at[pl.ds(c * NPAD + s * RPT, RPT)])


_deg_call = pl.kernel(
    _deg_body,
    out_type=jax.ShapeDtypeStruct((2 * NPAD,), jnp.float32),
    mesh=_mesh,
    scratch_types=[
        pltpu.VMEM((NCH, CHUNK), jnp.int32),
        pltpu.VMEM((CHUNK,), jnp.float32),
        pltpu.VMEM((CHUNK,), jnp.float32),
        pltpu.VMEM_SHARED((NPAD,), jnp.float32),
        pltpu.SemaphoreType.DMA,
    ],
    compiler_params=_sc_params,
)


def _msg_body(tab_hbm, ei_hbm, out_hbm, idx_s2, idx_d2, rows0, rows1, rows2,
              acc_sh, semi, sg0, sg1, sg2, ssc):
    c = lax.axis_index("c")
    s = lax.axis_index("s")
    wid = c * NS + s
    stg = rows2.at[pl.ds(0, RPT)]

    def zfill(i, _):
        rows2[i, :] = jnp.zeros((16,), jnp.float32)
        return 0

    lax.fori_loop(0, RPT, zfill, 0)
    descs = []
    for j in range(NCH):
        base = wid * EPT + j * CHUNK
        descs.append(pltpu.async_copy(
            ei_hbm.at[pl.ds(base, CHUNK)], idx_s2.at[j], semi))
        descs.append(pltpu.async_copy(
            ei_hbm.at[pl.ds(E + base, CHUNK)], idx_d2.at[j], semi))
    pltpu.sync_copy(stg, acc_sh.at[pl.ds(s * RPT, RPT)])
    descs[0].wait()
    _to_vorder(idx_s2, 0)
    for d in descs[1:]:
        d.wait()
    plsc.subcore_barrier()

    bufs = (rows0, rows1, rows2)
    sems = (sg0, sg1, sg2)
    g = [None] * NCH
    sc = [None] * NCH
    g[0] = pltpu.async_copy(tab_hbm.at[idx_s2.at[0]], bufs[0], sems[0])
    _to_vorder(idx_d2, 0)
    for j in range(NCH):
        if j + 1 < NCH:
            _to_vorder(idx_s2, j + 1)
            _to_vorder(idx_d2, j + 1)
        g[j].wait()
        if j >= 2:
            sc[j - 2].wait()
        if j + 1 < NCH:
            g[j + 1] = pltpu.async_copy(
                tab_hbm.at[idx_s2.at[j + 1]], bufs[(j + 1) % 3], sems[(j + 1) % 3])
        sc[j] = pltpu.async_copy(
            bufs[j % 3], acc_sh.at[idx_d2.at[j]], ssc, add=True)
    sc[NCH - 2].wait()
    sc[NCH - 1].wait()
    plsc.subcore_barrier()
    pltpu.sync_copy(acc_sh.at[pl.ds(s * RPT, RPT)], stg)
    pltpu.sync_copy(stg, out_hbm.at[pl.ds(c * NPAD + s * RPT, RPT)])


_msg_call = pl.kernel(
    _msg_body,
    out_type=jax.ShapeDtypeStruct((2 * NPAD, HID), jnp.float32),
    mesh=_mesh,
    scratch_types=[
        pltpu.VMEM((NCH, CHUNK), jnp.int32),
        pltpu.VMEM((NCH, CHUNK), jnp.int32),
        pltpu.VMEM((CHUNK, HID), jnp.float32),
        pltpu.VMEM((CHUNK, HID), jnp.float32),
        pltpu.VMEM((CHUNK, HID), jnp.float32),
        pltpu.VMEM_SHARED((NPAD, HID), jnp.float32),
        pltpu.SemaphoreType.DMA,
        pltpu.SemaphoreType.DMA,
        pltpu.SemaphoreType.DMA,
        pltpu.SemaphoreType.DMA,
        pltpu.SemaphoreType.DMA,
    ],
    compiler_params=_sc_params,
)


GRID = 2
RBP = GSZ // GRID
DBP = NPAD // 128 // GRID
PKR = GSZ
DEGR = NPAD // 128


def _dscale_blk(dp_ref, rbp, dbp):
    deg = dp_ref[0] + dp_ref[1] + 1.0
    dinv = lax.rsqrt(deg)
    r_col = lax.broadcasted_iota(jnp.int32, (rbp, dbp), 0)
    q_row = lax.broadcasted_iota(jnp.int32, (rbp, dbp), 1)
    qoh = jnp.where(r_col // 16 == q_row, 1.0, 0.0)
    u = jnp.dot(qoh, dinv, preferred_element_type=jnp.float32)
    r2 = lax.broadcasted_iota(jnp.int32, (rbp, 128), 0)
    l2 = lax.broadcasted_iota(jnp.int32, (rbp, 128), 1)
    cols = []
    for a in range(8):
        loh = jnp.where(l2 == 8 * (r2 % 16) + a, 1.0, 0.0)
        cols.append(jnp.sum(u * loh, axis=1, keepdims=True))
    dvec8 = jnp.concatenate(cols, axis=1)
    a_row = lax.broadcasted_iota(jnp.int32, (8, 128), 0)
    m_col = lax.broadcasted_iota(jnp.int32, (8, 128), 1)
    bcast = jnp.where(m_col // 16 == a_row, 1.0, 0.0)
    return jnp.dot(dvec8, bcast, preferred_element_type=jnp.float32)


def _s1a_body(x_ref, w_ref, o_ref):
    for a in range(8):
        rows = min(GSZ, N - GSZ * a)
        h = jnp.dot(x_ref[pl.ds(GSZ * a, rows), :], w_ref[...],
                    preferred_element_type=jnp.float32)
        o_ref[pl.ds(0, rows), pl.ds(a * HID, HID)] = h


def _s1b_body(h_ref, dp_ref, o_ref, ds_ref):
    dscale = _dscale_blk(dp_ref, RBP, DBP)
    ds_ref[...] = dscale
    o_ref[...] = h_ref[...] * dscale


def _s2_body(p_ref, h1s_ref, ds_ref, b1_ref, w2bd_ref, o_ref):
    dscale = ds_ref[...]
    m = p_ref[0] + p_ref[1] + h1s_ref[...]
    a = jnp.maximum(m * dscale + b1_ref[...], 0.0)
    h2 = jnp.dot(a, w2bd_ref[...], preferred_element_type=jnp.float32)
    o_ref[...] = h2 * dscale


def _s3_body(p_ref, h2s_ref, ds_ref, b2_ref, o_ref):
    mcol = lax.broadcasted_iota(jnp.int32, (PKR, HID), 1)
    mask = mcol < OUT_DIM
    ps = (p_ref[0] + p_ref[1] + h2s_ref[...]) * ds_ref[...]
    for a in range(8):
        z = ps[:, a * HID:(a + 1) * HID] + b2_ref[...]
        zm = jnp.where(mask, z, jnp.float32(-1e30))
        mx = jnp.max(zm, axis=1, keepdims=True)
        e = jnp.where(mask, jnp.exp(z - mx), 0.0)
        lse = jnp.log(jnp.sum(e, axis=1, keepdims=True)) + mx
        rows = min(GSZ, N - GSZ * a)
        o_ref[pl.ds(GSZ * a, rows), :] = (z - lse)[:rows, :OUT_DIM]


def kernel(x, edge_index, W1, b1, W2, b2):
    ei = edge_index.reshape(2 * E).astype(jnp.int32)
    W2p = jnp.pad(W2, ((0, 0), (0, HID - OUT_DIM)))
    aeq = (jnp.arange(128)[:, None] // HID) == (jnp.arange(128)[None, :] // HID)
    W2bd = jnp.where(aeq, jnp.tile(W2p, (8, 8)), 0.0)
    b1r = jnp.reshape(jnp.tile(b1, 8), (1, 128))
    b2p = jnp.reshape(jnp.pad(b2, (0, HID - OUT_DIM)), (1, HID))

    deg_pk = _deg_call(ei).reshape(2, DEGR, 128)

    h1_pk = pl.pallas_call(
        _s1a_body,
        in_specs=[
            pl.BlockSpec((N, IN_DIM), lambda: (0, 0)),
            pl.BlockSpec((IN_DIM, HID), lambda: (0, 0)),
        ],
        out_specs=pl.BlockSpec((GSZ, 128), lambda: (0, 0)),
        out_shape=jax.ShapeDtypeStruct((GSZ, 128), jnp.float32),
    )(x, W1)

    h1s_pk, dscale_pk = pl.pallas_call(
        _s1b_body,
        grid=(GRID,),
        in_specs=[
            pl.BlockSpec((RBP, 128), lambda i: (i, 0)),
            pl.BlockSpec((2, DBP, 128), lambda i: (0, i, 0)),
        ],
        out_specs=[
            pl.BlockSpec((RBP, 128), lambda i: (i, 0)),
            pl.BlockSpec((RBP, 128), lambda i: (i, 0)),
        ],
        out_shape=[
            jax.ShapeDtypeStruct((GSZ, 128), jnp.float32),
            jax.ShapeDtypeStruct((GSZ, 128), jnp.float32),
        ],
    )(h1_pk, deg_pk)

    p1_pk = _msg_call(h1s_pk.reshape(NPAD, HID), ei).reshape(2, GSZ, 128)

    h2s_pk = pl.pallas_call(
        _s2_body,
        grid=(GRID,),
        in_specs=[
            pl.BlockSpec((2, RBP, 128), lambda i: (0, i, 0)),
            pl.BlockSpec((RBP, 128), lambda i: (i, 0)),
            pl.BlockSpec((RBP, 128), lambda i: (i, 0)),
            pl.BlockSpec((1, 128), lambda i: (0, 0)),
            pl.BlockSpec((128, 128), lambda i: (0, 0)),
        ],
        out_specs=pl.BlockSpec((RBP, 128), lambda i: (i, 0)),
        out_shape=jax.ShapeDtypeStruct((GSZ, 128), jnp.float32),
    )(p1_pk, h1s_pk, dscale_pk, b1r, W2bd)

    p2_pk = _msg_call(h2s_pk.reshape(NPAD, HID), ei).reshape(2, GSZ, 128)

    out = pl.pallas_call(
        _s3_body,
        in_specs=[
            pl.BlockSpec((2, GSZ, 128), lambda: (0, 0, 0)),
            pl.BlockSpec((GSZ, 128), lambda: (0, 0)),
            pl.BlockSpec((GSZ, 128), lambda: (0, 0)),
            pl.BlockSpec((1, HID), lambda: (0, 0)),
        ],
        out_specs=pl.BlockSpec((N, OUT_DIM), lambda: (0, 0)),
        out_shape=jax.ShapeDtypeStruct((N, OUT_DIM), jnp.float32),
    )(p2_pk, h2s_pk, dscale_pk, b2p)

    return out
